# baseline (speedup 1.0000x reference)
import jax
import jax.numpy as jnp
from jax.experimental import pallas as pl
from jax.experimental.pallas import tpu as pltpu


_CHUNKS = 16


def _copy_body(img_in, txt_in, img_out, txt_out, sems):
    rows = img_in.shape[0] // _CHUNKS
    copies = []
    for i in range(_CHUNKS):
        sl = pl.ds(i * rows, rows)
        copies.append(
            pltpu.make_async_copy(img_in.at[sl], img_out.at[sl], sems.at[0, i])
        )
        copies.append(
            pltpu.make_async_copy(txt_in.at[sl], txt_out.at[sl], sems.at[1, i])
        )
    for c in copies:
        c.start()
    for c in copies:
        c.wait()


def kernel(image_feat, text_feat):
    out = pl.pallas_call(
        _copy_body,
        in_specs=[
            pl.BlockSpec(memory_space=pl.MemorySpace.ANY),
            pl.BlockSpec(memory_space=pl.MemorySpace.ANY),
        ],
        out_specs=[
            pl.BlockSpec(memory_space=pl.MemorySpace.ANY),
            pl.BlockSpec(memory_space=pl.MemorySpace.ANY),
        ],
        out_shape=[
            jax.ShapeDtypeStruct(image_feat.shape, image_feat.dtype),
            jax.ShapeDtypeStruct(text_feat.shape, text_feat.dtype),
        ],
        scratch_shapes=[pltpu.SemaphoreType.DMA((2, _CHUNKS))],
    )(image_feat, text_feat)
    return (out[0], out[1])


# speedup vs baseline: 47.9447x; 47.9447x over previous
import jax
import jax.numpy as jnp
from jax.experimental import pallas as pl
from jax.experimental.pallas import tpu as pltpu

_BLOCK_ROWS = 512


def _copy_body(img_in, txt_in, img_out, txt_out):
    img_out[...] = img_in[...]
    txt_out[...] = txt_in[...]


def kernel(image_feat, text_feat):
    n_rows, n_cols = image_feat.shape
    grid = (n_rows // _BLOCK_ROWS,)
    spec = pl.BlockSpec((_BLOCK_ROWS, n_cols), lambda i: (i, 0))
    out = pl.pallas_call(
        _copy_body,
        grid=grid,
        in_specs=[spec, spec],
        out_specs=[spec, spec],
        out_shape=[
            jax.ShapeDtypeStruct(image_feat.shape, image_feat.dtype),
            jax.ShapeDtypeStruct(text_feat.shape, text_feat.dtype),
        ],
    )(image_feat, text_feat)
    return (out[0], out[1])


# speedup vs baseline: 48.7431x; 1.0167x over previous
import jax
import jax.numpy as jnp
from jax.experimental import pallas as pl
from jax.experimental.pallas import tpu as pltpu

_BLOCK_ROWS = 1024


def _copy_body(img_in, txt_in, img_out, txt_out):
    img_out[...] = img_in[...]
    txt_out[...] = txt_in[...]


def kernel(image_feat, text_feat):
    n_rows, n_cols = image_feat.shape
    grid = (n_rows // _BLOCK_ROWS,)
    spec = pl.BlockSpec((_BLOCK_ROWS, n_cols), lambda i: (i, 0))
    out = pl.pallas_call(
        _copy_body,
        grid=grid,
        in_specs=[spec, spec],
        out_specs=[spec, spec],
        out_shape=[
            jax.ShapeDtypeStruct(image_feat.shape, image_feat.dtype),
            jax.ShapeDtypeStruct(text_feat.shape, text_feat.dtype),
        ],
    )(image_feat, text_feat)
    return (out[0], out[1])
